# baseline (speedup 1.0000x reference)
import functools

import jax
import jax.numpy as jnp
from jax import lax
from jax.experimental import pallas as pl
from jax.experimental.pallas import tpu as pltpu
from jax.experimental.pallas import tpu_sc as plsc

N = 10000
E = 320000
D = 128
C = 10
K1 = 8000
K2 = 6400
NP = 10240
NR = NP // 128
NC = 2
NS = 16
NW = NC * NS
CH = 79
CH0 = 48
CH1 = 112
CHT = CH0 + CH1
EPAD = NS * CHT * 128
RPT = NP // NS

_MESH = plsc.VectorSubcoreMesh(core_axis_name="c", subcore_axis_name="s",
                               num_cores=NC, num_subcores=NS)


@functools.partial(
    pl.kernel,
    out_type=jax.ShapeDtypeStruct((NC, NP, D), jnp.float32),
    mesh=_MESH,
    scratch_types=[
        pltpu.VMEM((CH1, 128), jnp.int32),
        pltpu.VMEM((CH1, 128), jnp.int32),
        pltpu.VMEM((128, D), jnp.float32),
        pltpu.VMEM_SHARED((NP, D), jnp.float32),
        pltpu.SemaphoreType.DMA,
    ],
)
def _sc_row_segsum(src_h, dst_h, y_h, out_h, srcv, dstv, rows, acc, sem):
    c = lax.axis_index("c")
    s = lax.axis_index("s")
    off = pl.multiple_of(c * CH0, 8)
    mych = jnp.where(c == 0, CH0, CH1)

    def zrow(i, _):
        rows[i // 8, pl.ds((i % 8) * 16, 16)] = jnp.zeros((16,), jnp.float32)
        return 0

    lax.fori_loop(0, 128 * (D // 16), zrow, 0)
    for t in range(RPT // 128):
        pltpu.sync_copy(rows, acc.at[pl.ds(s * RPT + t * 128, 128)])
    plsc.subcore_barrier()

    pltpu.sync_copy(src_h.at[s, pl.ds(off, CH1)], srcv)
    pltpu.sync_copy(dst_h.at[s, pl.ds(off, CH1)], dstv)

    def step(j, _):
        pltpu.async_copy(y_h.at[srcv.at[j]], rows, sem).wait()
        pltpu.sync_copy(rows, acc.at[dstv.at[j]], add=True)
        return 0

    lax.fori_loop(0, mych, step, 0)
    plsc.subcore_barrier()
    pltpu.sync_copy(acc.at[pl.ds(s * RPT, RPT)], out_h.at[c, pl.ds(s * RPT, RPT)])


@functools.partial(
    pl.kernel,
    out_type=jax.ShapeDtypeStruct((NC, NP), jnp.float32),
    mesh=_MESH,
    scratch_types=[
        pltpu.VMEM((CH1, 128), jnp.int32),
        pltpu.VMEM((CH1, 128), jnp.int32),
        pltpu.VMEM((128,), jnp.float32),
        pltpu.VMEM((128,), jnp.float32),
        pltpu.VMEM_SHARED((NP,), jnp.float32),
        pltpu.SemaphoreType.DMA,
    ],
)
def _sc_segsum(src_h, dst_h, val_h, out_h, srcv, dstv, vals, zbuf, acc, sem):
    c = lax.axis_index("c")
    s = lax.axis_index("s")
    off = pl.multiple_of(c * CH0, 8)
    mych = jnp.where(c == 0, CH0, CH1)

    def z16(i, _):
        zbuf[pl.ds(i * 16, 16)] = jnp.zeros((16,), jnp.float32)
        return 0

    lax.fori_loop(0, 8, z16, 0)
    for t in range(RPT // 128):
        pltpu.sync_copy(zbuf, acc.at[pl.ds(s * RPT + t * 128, 128)])
    plsc.subcore_barrier()

    pltpu.sync_copy(src_h.at[s, pl.ds(off, CH1)], srcv)
    pltpu.sync_copy(dst_h.at[s, pl.ds(off, CH1)], dstv)

    def step(j, _):
        pltpu.async_copy(val_h.at[srcv.at[j]], vals, sem).wait()
        pltpu.sync_copy(vals, acc.at[dstv.at[j]], add=True)
        return 0

    lax.fori_loop(0, mych, step, 0)
    plsc.subcore_barrier()
    pltpu.sync_copy(acc.at[pl.ds(s * RPT, RPT)], out_h.at[c, pl.ds(s * RPT, RPT)])


@functools.partial(
    pl.kernel,
    out_type=jax.ShapeDtypeStruct((NC, NP), jnp.float32),
    mesh=_MESH,
    scratch_types=[
        pltpu.VMEM((CH1, 128), jnp.int32),
        pltpu.VMEM((128,), jnp.float32),
        pltpu.VMEM((128,), jnp.float32),
        pltpu.VMEM_SHARED((NP,), jnp.float32),
    ],
)
def _sc_count(dst_h, out_h, dstv, ones, zbuf, acc):
    c = lax.axis_index("c")
    s = lax.axis_index("s")
    off = pl.multiple_of(c * CH0, 8)
    mych = jnp.where(c == 0, CH0, CH1)

    def init16(i, _):
        zbuf[pl.ds(i * 16, 16)] = jnp.zeros((16,), jnp.float32)
        ones[pl.ds(i * 16, 16)] = jnp.ones((16,), jnp.float32)
        return 0

    lax.fori_loop(0, 8, init16, 0)
    for t in range(RPT // 128):
        pltpu.sync_copy(zbuf, acc.at[pl.ds(s * RPT + t * 128, 128)])
    plsc.subcore_barrier()

    pltpu.sync_copy(dst_h.at[s, pl.ds(off, CH1)], dstv)

    def step(j, _):
        pltpu.sync_copy(ones, acc.at[dstv.at[j]], add=True)
        return 0

    lax.fori_loop(0, mych, step, 0)
    plsc.subcore_barrier()
    pltpu.sync_copy(acc.at[pl.ds(s * RPT, RPT)], out_h.at[c, pl.ds(s * RPT, RPT)])


def _topk_keep(score2d, k):
    b = lax.bitcast_convert_type(score2d, jnp.int32)
    key = b ^ ((b >> 31) & jnp.int32(0x7FFFFFFF))

    def body(i, t):
        cand = t + lax.shift_left(jnp.int32(1), 31 - i)
        cnt = jnp.sum((key >= cand).astype(jnp.int32))
        return jnp.where(cnt >= k, cand, t)

    t = lax.fori_loop(0, 32, body, jnp.int32(-2**31))
    gt = key > t
    eq = key == t
    eqf = eq.astype(jnp.float32)
    need = (jnp.int32(k) - jnp.sum(gt.astype(jnp.int32))).astype(jnp.float32)
    rows = jnp.sum(eqf, axis=1, keepdims=True)
    tri = (lax.broadcasted_iota(jnp.int32, (NR, NR), 0)
           > lax.broadcasted_iota(jnp.int32, (NR, NR), 1)).astype(jnp.float32)
    excl_rows = jnp.dot(tri, rows, preferred_element_type=jnp.float32)
    ut = (lax.broadcasted_iota(jnp.int32, (128, 128), 0)
          < lax.broadcasted_iota(jnp.int32, (128, 128), 1)).astype(jnp.float32)
    prefix = excl_rows + jnp.dot(eqf, ut, preferred_element_type=jnp.float32)
    return jnp.where(gt | (eq & (prefix < need)), 1.0, 0.0).astype(jnp.float32)


def _tc_scale(degp_ref, x_ref, w_ref, y_ref, dinv_ref):
    deg = degp_ref[0] + degp_ref[1] + 1.0
    dinv = lax.rsqrt(deg)
    xw = jnp.dot(x_ref[...], w_ref[...], preferred_element_type=jnp.float32)
    y_ref[...] = dinv * xw
    dinv_ref[...] = dinv


def _tc_scale2(kp_ref, keep_ref, h1_ref, w_ref, y_ref, dinv_ref):
    keep = keep_ref[...]
    deg = keep * (kp_ref[0] + kp_ref[1]) + 1.0
    dinv = lax.rsqrt(deg)
    xw = jnp.dot(h1_ref[...], w_ref[...], preferred_element_type=jnp.float32)
    y_ref[...] = (dinv * keep) * xw
    dinv_ref[...] = dinv


def _tc_conv_score(rsp_ref, y_ref, dinv_ref, b_ref, wn_ref, wr_ref, keep_ref,
                   h_ref, sn_ref, sr_ref):
    h = jnp.maximum(
        dinv_ref[...] * (rsp_ref[0] + rsp_ref[1] + y_ref[...]) + b_ref[...], 0.0)
    h_ref[...] = h
    sn_ref[...] = keep_ref[...] * jnp.dot(h, wn_ref[...],
                                          preferred_element_type=jnp.float32)
    sr_ref[...] = jnp.dot(h, wr_ref[...], preferred_element_type=jnp.float32)


def _tc_pool(k):
  def body(segp_ref, sr_ref, pb_ref, keep_in_ref, h_ref,
           h1_ref, keep_ref, xr_ref):
    score = segp_ref[0] + segp_ref[1] + sr_ref[...] + pb_ref[...]
    valid = keep_in_ref[...] > 0.0
    score = jnp.where(valid, score, -jnp.inf)
    score2d = score.reshape(NR, 128)
    keep2d = _topk_keep(score2d, k)
    gate = (keep2d * jnp.tanh(score2d)).reshape(NP, 1)
    keep = keep2d.reshape(NP, 1)
    h1 = h_ref[...] * gate
    h1_ref[...] = h1
    keep_ref[...] = keep
    mx = jnp.max(jnp.where(keep > 0.0, h1, -jnp.inf), axis=0)
    mn = jnp.sum(h1, axis=0) * (1.0 / k)
    xr_ref[...] = jnp.concatenate([mx, mn])[None, :]
  return body


def _tc_head(x1_ref, x2_ref, fw1_ref, fb1_ref, fw2_ref, fb2_ref, out_ref):
    z = x1_ref[...] + x2_ref[...]
    z = jnp.maximum(
        jnp.dot(z, fw1_ref[...], preferred_element_type=jnp.float32)
        + fb1_ref[...], 0.0)
    z = jnp.dot(z, fw2_ref[...], preferred_element_type=jnp.float32) + fb2_ref[...]
    m = jnp.max(z, axis=1, keepdims=True)
    out_ref[...] = (z - m) - jnp.log(jnp.sum(jnp.exp(z - m), axis=1, keepdims=True))


def _call(body, out_shapes, *args):
    return pl.pallas_call(body, out_shape=out_shapes)(*args)


def kernel(x, edge_index, batch, W1, b1, pWn1, pWr1, pb1, W2, b2, pWn2, pWr2,
           pb2, fW1, fb1, fW2, fb2):
    del batch
    f32 = jnp.float32
    sds = jax.ShapeDtypeStruct

    src = edge_index[0].astype(jnp.int32)
    dst = edge_index[1].astype(jnp.int32)
    pad_e = EPAD - E
    srcp = jnp.concatenate([src, jnp.zeros((pad_e,), jnp.int32)]).reshape(NS, CHT, 128)
    dstp = jnp.concatenate([dst, jnp.full((pad_e,), N, jnp.int32)]).reshape(NS, CHT, 128)
    xp = jnp.concatenate([x, jnp.zeros((NP - N, D), f32)], axis=0)
    allv = jnp.concatenate(
        [jnp.ones((N, 1), f32), jnp.zeros((NP - N, 1), f32)], axis=0)

    degp = _sc_count(dstp)
    y1, dinv1 = _call(
        _tc_scale, (sds((NP, D), f32), sds((NP, 1), f32)),
        degp.reshape(NC, NP, 1), xp, W1)
    rsp1 = _sc_row_segsum(srcp, dstp, y1)
    h, sn1, sr1 = _call(
        _tc_conv_score,
        (sds((NP, D), f32), sds((NP, 1), f32), sds((NP, 1), f32)),
        rsp1, y1, dinv1, b1.reshape(1, D), pWn1, pWr1, allv)
    segp1 = _sc_segsum(srcp, dstp, sn1.reshape(NP))
    h1, keep1, x1 = _call(
        _tc_pool(K1),
        (sds((NP, D), f32), sds((NP, 1), f32), sds((1, 2 * D), f32)),
        segp1.reshape(NC, NP, 1), sr1, pb1.reshape(1, 1), allv, h)

    kp = _sc_segsum(srcp, dstp, keep1.reshape(NP))
    y2, dinv2 = _call(
        _tc_scale2, (sds((NP, D), f32), sds((NP, 1), f32)),
        kp.reshape(NC, NP, 1), keep1, h1, W2)
    rsp2 = _sc_row_segsum(srcp, dstp, y2)
    h2, t2, sr2 = _call(
        _tc_conv_score,
        (sds((NP, D), f32), sds((NP, 1), f32), sds((NP, 1), f32)),
        rsp2, y2, dinv2, b2.reshape(1, D), pWn2, pWr2, keep1)
    segp2 = _sc_segsum(srcp, dstp, t2.reshape(NP))
    _, _, x2 = _call(
        _tc_pool(K2),
        (sds((NP, D), f32), sds((NP, 1), f32), sds((1, 2 * D), f32)),
        segp2.reshape(NC, NP, 1), sr2, pb2.reshape(1, 1), keep1, h2)

    return _call(_tc_head, sds((1, C), f32), x1, x2, fW1, fb1.reshape(1, D),
                 fW2, fb2.reshape(1, C))


# speedup vs baseline: 1.6188x; 1.6188x over previous
import functools

import jax
import jax.numpy as jnp
from jax import lax
from jax.experimental import pallas as pl
from jax.experimental.pallas import tpu as pltpu
from jax.experimental.pallas import tpu_sc as plsc

N = 10000
E = 320000
D = 128
C = 10
K1 = 8000
K2 = 6400
NP = 10240
NR = NP // 128
NC = 2
NS = 16
NW = NC * NS
CH = 79
EPAD = NW * CH * 128
RPT = NP // NS

_MESH = plsc.VectorSubcoreMesh(core_axis_name="c", subcore_axis_name="s",
                               num_cores=NC, num_subcores=NS)


@functools.partial(
    pl.kernel,
    out_type=jax.ShapeDtypeStruct((NC, NP, D), jnp.float32),
    mesh=_MESH,
    scratch_types=[
        pltpu.VMEM((CH, 128), jnp.int32),
        pltpu.VMEM((CH, 128), jnp.int32),
        pltpu.VMEM((128, D), jnp.float32),
        pltpu.VMEM_SHARED((NP, D), jnp.float32),
        pltpu.SemaphoreType.DMA,
    ],
)
def _sc_row_segsum(src_h, dst_h, y_h, out_h, srcv, dstv, rows, acc, sem):
    c = lax.axis_index("c")
    s = lax.axis_index("s")
    wid = s * NC + c

    def zrow(i, _):
        rows[i // 8, pl.ds((i % 8) * 16, 16)] = jnp.zeros((16,), jnp.float32)
        return 0

    lax.fori_loop(0, 128 * (D // 16), zrow, 0)
    for t in range(RPT // 128):
        pltpu.sync_copy(rows, acc.at[pl.ds(s * RPT + t * 128, 128)])
    plsc.subcore_barrier()

    pltpu.sync_copy(src_h.at[wid], srcv)
    pltpu.sync_copy(dst_h.at[wid], dstv)

    def step(j, _):
        pltpu.async_copy(y_h.at[srcv.at[j]], rows, sem).wait()
        pltpu.sync_copy(rows, acc.at[dstv.at[j]], add=True)
        return 0

    lax.fori_loop(0, CH, step, 0)
    plsc.subcore_barrier()
    pltpu.sync_copy(acc.at[pl.ds(s * RPT, RPT)], out_h.at[c, pl.ds(s * RPT, RPT)])


@functools.partial(
    pl.kernel,
    out_type=jax.ShapeDtypeStruct((NC, NP), jnp.float32),
    mesh=_MESH,
    scratch_types=[
        pltpu.VMEM((CH, 128), jnp.int32),
        pltpu.VMEM((CH, 128), jnp.int32),
        pltpu.VMEM((128,), jnp.float32),
        pltpu.VMEM((128,), jnp.float32),
        pltpu.VMEM_SHARED((NP,), jnp.float32),
        pltpu.SemaphoreType.DMA,
    ],
)
def _sc_segsum(src_h, dst_h, val_h, out_h, srcv, dstv, vals, zbuf, acc, sem):
    c = lax.axis_index("c")
    s = lax.axis_index("s")
    wid = s * NC + c

    def z16(i, _):
        zbuf[pl.ds(i * 16, 16)] = jnp.zeros((16,), jnp.float32)
        return 0

    lax.fori_loop(0, 8, z16, 0)
    for t in range(RPT // 128):
        pltpu.sync_copy(zbuf, acc.at[pl.ds(s * RPT + t * 128, 128)])
    plsc.subcore_barrier()

    pltpu.sync_copy(src_h.at[wid], srcv)
    pltpu.sync_copy(dst_h.at[wid], dstv)

    def step(j, _):
        pltpu.async_copy(val_h.at[srcv.at[j]], vals, sem).wait()
        pltpu.sync_copy(vals, acc.at[dstv.at[j]], add=True)
        return 0

    lax.fori_loop(0, CH, step, 0)
    plsc.subcore_barrier()
    pltpu.sync_copy(acc.at[pl.ds(s * RPT, RPT)], out_h.at[c, pl.ds(s * RPT, RPT)])


@functools.partial(
    pl.kernel,
    out_type=jax.ShapeDtypeStruct((NC, NP), jnp.float32),
    mesh=_MESH,
    scratch_types=[
        pltpu.VMEM((CH, 128), jnp.int32),
        pltpu.VMEM((128,), jnp.float32),
        pltpu.VMEM((128,), jnp.float32),
        pltpu.VMEM_SHARED((NP,), jnp.float32),
    ],
)
def _sc_count(dst_h, out_h, dstv, ones, zbuf, acc):
    c = lax.axis_index("c")
    s = lax.axis_index("s")
    wid = s * NC + c

    def init16(i, _):
        zbuf[pl.ds(i * 16, 16)] = jnp.zeros((16,), jnp.float32)
        ones[pl.ds(i * 16, 16)] = jnp.ones((16,), jnp.float32)
        return 0

    lax.fori_loop(0, 8, init16, 0)
    for t in range(RPT // 128):
        pltpu.sync_copy(zbuf, acc.at[pl.ds(s * RPT + t * 128, 128)])
    plsc.subcore_barrier()

    pltpu.sync_copy(dst_h.at[wid], dstv)

    def step(j, _):
        pltpu.sync_copy(ones, acc.at[dstv.at[j]], add=True)
        return 0

    lax.fori_loop(0, CH, step, 0)
    plsc.subcore_barrier()
    pltpu.sync_copy(acc.at[pl.ds(s * RPT, RPT)], out_h.at[c, pl.ds(s * RPT, RPT)])


def _topk_keep(score2d, k):
    b = lax.bitcast_convert_type(score2d, jnp.int32)
    key = b ^ ((b >> 31) & jnp.int32(0x7FFFFFFF))

    def body(i, t):
        cand = t + lax.shift_left(jnp.int32(1), 31 - i)
        cnt = jnp.sum((key >= cand).astype(jnp.int32))
        return jnp.where(cnt >= k, cand, t)

    t = lax.fori_loop(0, 32, body, jnp.int32(-2**31))
    gt = key > t
    eq = key == t
    eqf = eq.astype(jnp.float32)
    need = (jnp.int32(k) - jnp.sum(gt.astype(jnp.int32))).astype(jnp.float32)
    rows = jnp.sum(eqf, axis=1, keepdims=True)
    tri = (lax.broadcasted_iota(jnp.int32, (NR, NR), 0)
           > lax.broadcasted_iota(jnp.int32, (NR, NR), 1)).astype(jnp.float32)
    excl_rows = jnp.dot(tri, rows, preferred_element_type=jnp.float32)
    ut = (lax.broadcasted_iota(jnp.int32, (128, 128), 0)
          < lax.broadcasted_iota(jnp.int32, (128, 128), 1)).astype(jnp.float32)
    prefix = excl_rows + jnp.dot(eqf, ut, preferred_element_type=jnp.float32)
    return jnp.where(gt | (eq & (prefix < need)), 1.0, 0.0).astype(jnp.float32)


def _tc_matmul(x_ref, w_ref, o_ref):
    o_ref[...] = jnp.dot(x_ref[...], w_ref[...],
                         preferred_element_type=jnp.float32)


def _tc_scale(degp_ref, xw_ref, y_ref, dinv_ref):
    deg = degp_ref[0] + degp_ref[1] + 1.0
    dinv = lax.rsqrt(deg)
    y_ref[...] = dinv * xw_ref[...]
    dinv_ref[...] = dinv


def _tc_scale2(kp_ref, keep_ref, xw_ref, y_ref, dinv_ref):
    keep = keep_ref[...]
    deg = keep * (kp_ref[0] + kp_ref[1]) + 1.0
    dinv = lax.rsqrt(deg)
    y_ref[...] = (dinv * keep) * xw_ref[...]
    dinv_ref[...] = dinv


def _tc_conv_score(rsp_ref, y_ref, dinv_ref, b_ref, wn_ref, wr_ref, keep_ref,
                   h_ref, sn_ref, sr_ref):
    h = jnp.maximum(
        dinv_ref[...] * (rsp_ref[0] + rsp_ref[1] + y_ref[...]) + b_ref[...], 0.0)
    h_ref[...] = h
    sn_ref[...] = keep_ref[...] * jnp.dot(h, wn_ref[...],
                                          preferred_element_type=jnp.float32)
    sr_ref[...] = jnp.dot(h, wr_ref[...], preferred_element_type=jnp.float32)


def _tc_pool(k):
  def body(segp_ref, sr_ref, pb_ref, keep_in_ref, h_ref,
           h1_ref, keep_ref, xr_ref):
    score = segp_ref[0] + segp_ref[1] + sr_ref[...] + pb_ref[...]
    valid = keep_in_ref[...] > 0.0
    score = jnp.where(valid, score, -jnp.inf)
    score2d = score.reshape(NR, 128)
    keep2d = _topk_keep(score2d, k)
    gate = (keep2d * jnp.tanh(score2d)).reshape(NP, 1)
    keep = keep2d.reshape(NP, 1)
    h1 = h_ref[...] * gate
    h1_ref[...] = h1
    keep_ref[...] = keep
    mx = jnp.max(jnp.where(keep > 0.0, h1, -jnp.inf), axis=0)
    mn = jnp.sum(h1, axis=0) * (1.0 / k)
    xr_ref[...] = jnp.concatenate([mx, mn])[None, :]
  return body


def _tc_head(x1_ref, x2_ref, fw1_ref, fb1_ref, fw2_ref, fb2_ref, out_ref):
    z = x1_ref[...] + x2_ref[...]
    z = jnp.maximum(
        jnp.dot(z, fw1_ref[...], preferred_element_type=jnp.float32)
        + fb1_ref[...], 0.0)
    z = jnp.dot(z, fw2_ref[...], preferred_element_type=jnp.float32) + fb2_ref[...]
    m = jnp.max(z, axis=1, keepdims=True)
    out_ref[...] = (z - m) - jnp.log(jnp.sum(jnp.exp(z - m), axis=1, keepdims=True))


def _call(body, out_shapes, *args):
    return pl.pallas_call(body, out_shape=out_shapes)(*args)


def kernel(x, edge_index, batch, W1, b1, pWn1, pWr1, pb1, W2, b2, pWn2, pWr2,
           pb2, fW1, fb1, fW2, fb2):
    del batch
    f32 = jnp.float32
    sds = jax.ShapeDtypeStruct

    src = edge_index[0].astype(jnp.int32)
    dst = edge_index[1].astype(jnp.int32)
    pad_e = EPAD - E
    srcp = jnp.concatenate([src, jnp.zeros((pad_e,), jnp.int32)]).reshape(NW, CH, 128)
    dstp = jnp.concatenate([dst, jnp.full((pad_e,), N, jnp.int32)]).reshape(NW, CH, 128)
    xp = jnp.concatenate([x, jnp.zeros((NP - N, D), f32)], axis=0)
    allv = jnp.concatenate(
        [jnp.ones((N, 1), f32), jnp.zeros((NP - N, 1), f32)], axis=0)

    degp = _sc_count(dstp)
    xw1 = _call(_tc_matmul, sds((NP, D), f32), xp, W1)
    y1, dinv1 = _call(
        _tc_scale, (sds((NP, D), f32), sds((NP, 1), f32)),
        degp.reshape(NC, NP, 1), xw1)
    rsp1 = _sc_row_segsum(srcp, dstp, y1)
    h, sn1, sr1 = _call(
        _tc_conv_score,
        (sds((NP, D), f32), sds((NP, 1), f32), sds((NP, 1), f32)),
        rsp1, y1, dinv1, b1.reshape(1, D), pWn1, pWr1, allv)
    segp1 = _sc_segsum(srcp, dstp, sn1.reshape(NP))
    h1, keep1, x1 = _call(
        _tc_pool(K1),
        (sds((NP, D), f32), sds((NP, 1), f32), sds((1, 2 * D), f32)),
        segp1.reshape(NC, NP, 1), sr1, pb1.reshape(1, 1), allv, h)

    kp = _sc_segsum(srcp, dstp, keep1.reshape(NP))
    xw2 = _call(_tc_matmul, sds((NP, D), f32), h1, W2)
    y2, dinv2 = _call(
        _tc_scale2, (sds((NP, D), f32), sds((NP, 1), f32)),
        kp.reshape(NC, NP, 1), keep1, xw2)
    rsp2 = _sc_row_segsum(srcp, dstp, y2)
    h2, t2, sr2 = _call(
        _tc_conv_score,
        (sds((NP, D), f32), sds((NP, 1), f32), sds((NP, 1), f32)),
        rsp2, y2, dinv2, b2.reshape(1, D), pWn2, pWr2, keep1)
    segp2 = _sc_segsum(srcp, dstp, t2.reshape(NP))
    _, _, x2 = _call(
        _tc_pool(K2),
        (sds((NP, D), f32), sds((NP, 1), f32), sds((1, 2 * D), f32)),
        segp2.reshape(NC, NP, 1), sr2, pb2.reshape(1, 1), keep1, h2)

    return _call(_tc_head, sds((1, C), f32), x1, x2, fW1, fb1.reshape(1, D),
                 fW2, fb2.reshape(1, C))


# speedup vs baseline: 1.6214x; 1.0016x over previous
import functools

import jax
import jax.numpy as jnp
from jax import lax
from jax.experimental import pallas as pl
from jax.experimental.pallas import tpu as pltpu
from jax.experimental.pallas import tpu_sc as plsc

N = 10000
E = 320000
D = 128
C = 10
K1 = 8000
K2 = 6400
NP = 10240
NR = NP // 128
NC = 2
NS = 16
NW = NC * NS
CH = 79
EPAD = NW * CH * 128
RPT = NP // NS

_MESH = plsc.VectorSubcoreMesh(core_axis_name="c", subcore_axis_name="s",
                               num_cores=NC, num_subcores=NS)


@functools.partial(
    pl.kernel,
    out_type=jax.ShapeDtypeStruct((NC, NP, D), jnp.float32),
    mesh=_MESH,
    scratch_types=[
        pltpu.VMEM((CH, 128), jnp.int32),
        pltpu.VMEM((CH, 128), jnp.int32),
        pltpu.VMEM((128, D), jnp.float32),
        pltpu.VMEM_SHARED((NP, D), jnp.float32),
        pltpu.SemaphoreType.DMA,
    ],
)
def _sc_row_segsum(src_h, dst_h, y_h, out_h, srcv, dstv, rows, acc, sem):
    c = lax.axis_index("c")
    s = lax.axis_index("s")
    wid = s * NC + c

    def zrow(i, _):
        rows[i // 8, pl.ds((i % 8) * 16, 16)] = jnp.zeros((16,), jnp.float32)
        return 0

    lax.fori_loop(0, 128 * (D // 16), zrow, 0)
    for t in range(RPT // 128):
        pltpu.sync_copy(rows, acc.at[pl.ds(s * RPT + t * 128, 128)])
    plsc.subcore_barrier()

    pltpu.sync_copy(src_h.at[wid], srcv)
    pltpu.sync_copy(dst_h.at[wid], dstv)

    def step(j, _):
        pltpu.async_copy(y_h.at[srcv.at[j]], rows, sem).wait()
        pltpu.sync_copy(rows, acc.at[dstv.at[j]], add=True)
        return 0

    lax.fori_loop(0, CH, step, 0)
    plsc.subcore_barrier()
    pltpu.sync_copy(acc.at[pl.ds(s * RPT, RPT)], out_h.at[c, pl.ds(s * RPT, RPT)])


@functools.partial(
    pl.kernel,
    out_type=jax.ShapeDtypeStruct((NC, NP), jnp.float32),
    mesh=_MESH,
    scratch_types=[
        pltpu.VMEM((CH, 128), jnp.int32),
        pltpu.VMEM((CH, 128), jnp.int32),
        pltpu.VMEM((128,), jnp.float32),
        pltpu.VMEM((128,), jnp.float32),
        pltpu.VMEM_SHARED((NP,), jnp.float32),
        pltpu.SemaphoreType.DMA,
    ],
)
def _sc_segsum(src_h, dst_h, val_h, out_h, srcv, dstv, vals, zbuf, acc, sem):
    c = lax.axis_index("c")
    s = lax.axis_index("s")
    wid = s * NC + c

    def z16(i, _):
        zbuf[pl.ds(i * 16, 16)] = jnp.zeros((16,), jnp.float32)
        return 0

    lax.fori_loop(0, 8, z16, 0)
    for t in range(RPT // 128):
        pltpu.sync_copy(zbuf, acc.at[pl.ds(s * RPT + t * 128, 128)])
    plsc.subcore_barrier()

    pltpu.sync_copy(src_h.at[wid], srcv)
    pltpu.sync_copy(dst_h.at[wid], dstv)

    def step(j, _):
        pltpu.async_copy(val_h.at[srcv.at[j]], vals, sem).wait()
        pltpu.sync_copy(vals, acc.at[dstv.at[j]], add=True)
        return 0

    lax.fori_loop(0, CH, step, 0)
    plsc.subcore_barrier()
    pltpu.sync_copy(acc.at[pl.ds(s * RPT, RPT)], out_h.at[c, pl.ds(s * RPT, RPT)])


@functools.partial(
    pl.kernel,
    out_type=jax.ShapeDtypeStruct((NC, NP), jnp.float32),
    mesh=_MESH,
    scratch_types=[
        pltpu.VMEM((CH, 128), jnp.int32),
        pltpu.VMEM((128,), jnp.float32),
        pltpu.VMEM((128,), jnp.float32),
        pltpu.VMEM_SHARED((NP,), jnp.float32),
        pltpu.SemaphoreType.DMA,
    ],
)
def _sc_count(dst_h, out_h, dstv, ones, zbuf, acc, sem):
    c = lax.axis_index("c")
    s = lax.axis_index("s")
    wid = s * NC + c

    def init16(i, _):
        zbuf[pl.ds(i * 16, 16)] = jnp.zeros((16,), jnp.float32)
        ones[pl.ds(i * 16, 16)] = jnp.ones((16,), jnp.float32)
        return 0

    lax.fori_loop(0, 8, init16, 0)
    for t in range(RPT // 128):
        pltpu.sync_copy(zbuf, acc.at[pl.ds(s * RPT + t * 128, 128)])
    plsc.subcore_barrier()

    pltpu.sync_copy(dst_h.at[wid], dstv)

    def step(j, _):
        pltpu.async_copy(ones, acc.at[dstv.at[j]], sem, add=True)
        return 0

    lax.fori_loop(0, CH, step, 0)

    def drain(j, _):
        pltpu.make_async_copy(out_h.at[0, pl.ds(0, 128)], ones, sem).wait()
        return 0

    lax.fori_loop(0, CH, drain, 0)
    plsc.subcore_barrier()
    pltpu.sync_copy(acc.at[pl.ds(s * RPT, RPT)], out_h.at[c, pl.ds(s * RPT, RPT)])


def _topk_keep(score2d, k):
    b = lax.bitcast_convert_type(score2d, jnp.int32)
    key = b ^ ((b >> 31) & jnp.int32(0x7FFFFFFF))

    def body(i, t):
        cand = t + lax.shift_left(jnp.int32(1), 31 - i)
        cnt = jnp.sum((key >= cand).astype(jnp.int32))
        return jnp.where(cnt >= k, cand, t)

    t = lax.fori_loop(0, 32, body, jnp.int32(-2**31))
    gt = key > t
    eq = key == t
    eqf = eq.astype(jnp.float32)
    need = (jnp.int32(k) - jnp.sum(gt.astype(jnp.int32))).astype(jnp.float32)
    rows = jnp.sum(eqf, axis=1, keepdims=True)
    tri = (lax.broadcasted_iota(jnp.int32, (NR, NR), 0)
           > lax.broadcasted_iota(jnp.int32, (NR, NR), 1)).astype(jnp.float32)
    excl_rows = jnp.dot(tri, rows, preferred_element_type=jnp.float32)
    ut = (lax.broadcasted_iota(jnp.int32, (128, 128), 0)
          < lax.broadcasted_iota(jnp.int32, (128, 128), 1)).astype(jnp.float32)
    prefix = excl_rows + jnp.dot(eqf, ut, preferred_element_type=jnp.float32)
    return jnp.where(gt | (eq & (prefix < need)), 1.0, 0.0).astype(jnp.float32)


def _tc_matmul(x_ref, w_ref, o_ref):
    o_ref[...] = jnp.dot(x_ref[...], w_ref[...],
                         preferred_element_type=jnp.float32)


def _tc_scale(degp_ref, xw_ref, y_ref, dinv_ref):
    deg = degp_ref[0] + degp_ref[1] + 1.0
    dinv = lax.rsqrt(deg)
    y_ref[...] = dinv * xw_ref[...]
    dinv_ref[...] = dinv


def _tc_scale2(kp_ref, keep_ref, xw_ref, y_ref, dinv_ref):
    keep = keep_ref[...]
    deg = keep * (kp_ref[0] + kp_ref[1]) + 1.0
    dinv = lax.rsqrt(deg)
    y_ref[...] = (dinv * keep) * xw_ref[...]
    dinv_ref[...] = dinv


def _tc_conv_score(rsp_ref, y_ref, dinv_ref, b_ref, wn_ref, wr_ref, keep_ref,
                   h_ref, sn_ref, sr_ref):
    h = jnp.maximum(
        dinv_ref[...] * (rsp_ref[0] + rsp_ref[1] + y_ref[...]) + b_ref[...], 0.0)
    h_ref[...] = h
    sn_ref[...] = keep_ref[...] * jnp.dot(h, wn_ref[...],
                                          preferred_element_type=jnp.float32)
    sr_ref[...] = jnp.dot(h, wr_ref[...], preferred_element_type=jnp.float32)


def _tc_pool(k):
  def body(segp_ref, sr_ref, pb_ref, keep_in_ref, h_ref,
           h1_ref, keep_ref, xr_ref):
    score = segp_ref[0] + segp_ref[1] + sr_ref[...] + pb_ref[...]
    valid = keep_in_ref[...] > 0.0
    score = jnp.where(valid, score, -jnp.inf)
    score2d = score.reshape(NR, 128)
    keep2d = _topk_keep(score2d, k)
    gate = (keep2d * jnp.tanh(score2d)).reshape(NP, 1)
    keep = keep2d.reshape(NP, 1)
    h1 = h_ref[...] * gate
    h1_ref[...] = h1
    keep_ref[...] = keep
    mx = jnp.max(jnp.where(keep > 0.0, h1, -jnp.inf), axis=0)
    mn = jnp.sum(h1, axis=0) * (1.0 / k)
    xr_ref[...] = jnp.concatenate([mx, mn])[None, :]
  return body


def _tc_head(x1_ref, x2_ref, fw1_ref, fb1_ref, fw2_ref, fb2_ref, out_ref):
    z = x1_ref[...] + x2_ref[...]
    z = jnp.maximum(
        jnp.dot(z, fw1_ref[...], preferred_element_type=jnp.float32)
        + fb1_ref[...], 0.0)
    z = jnp.dot(z, fw2_ref[...], preferred_element_type=jnp.float32) + fb2_ref[...]
    m = jnp.max(z, axis=1, keepdims=True)
    out_ref[...] = (z - m) - jnp.log(jnp.sum(jnp.exp(z - m), axis=1, keepdims=True))


def _call(body, out_shapes, *args):
    return pl.pallas_call(body, out_shape=out_shapes)(*args)


def kernel(x, edge_index, batch, W1, b1, pWn1, pWr1, pb1, W2, b2, pWn2, pWr2,
           pb2, fW1, fb1, fW2, fb2):
    del batch
    f32 = jnp.float32
    sds = jax.ShapeDtypeStruct

    src = edge_index[0].astype(jnp.int32)
    dst = edge_index[1].astype(jnp.int32)
    pad_e = EPAD - E
    srcp = jnp.concatenate([src, jnp.zeros((pad_e,), jnp.int32)]).reshape(NW, CH, 128)
    dstp = jnp.concatenate([dst, jnp.full((pad_e,), N, jnp.int32)]).reshape(NW, CH, 128)
    xp = jnp.concatenate([x, jnp.zeros((NP - N, D), f32)], axis=0)
    allv = jnp.concatenate(
        [jnp.ones((N, 1), f32), jnp.zeros((NP - N, 1), f32)], axis=0)

    degp = _sc_count(dstp)
    xw1 = _call(_tc_matmul, sds((NP, D), f32), xp, W1)
    y1, dinv1 = _call(
        _tc_scale, (sds((NP, D), f32), sds((NP, 1), f32)),
        degp.reshape(NC, NP, 1), xw1)
    rsp1 = _sc_row_segsum(srcp, dstp, y1)
    h, sn1, sr1 = _call(
        _tc_conv_score,
        (sds((NP, D), f32), sds((NP, 1), f32), sds((NP, 1), f32)),
        rsp1, y1, dinv1, b1.reshape(1, D), pWn1, pWr1, allv)
    segp1 = _sc_segsum(srcp, dstp, sn1.reshape(NP))
    h1, keep1, x1 = _call(
        _tc_pool(K1),
        (sds((NP, D), f32), sds((NP, 1), f32), sds((1, 2 * D), f32)),
        segp1.reshape(NC, NP, 1), sr1, pb1.reshape(1, 1), allv, h)

    kp = _sc_segsum(srcp, dstp, keep1.reshape(NP))
    xw2 = _call(_tc_matmul, sds((NP, D), f32), h1, W2)
    y2, dinv2 = _call(
        _tc_scale2, (sds((NP, D), f32), sds((NP, 1), f32)),
        kp.reshape(NC, NP, 1), keep1, xw2)
    rsp2 = _sc_row_segsum(srcp, dstp, y2)
    h2, t2, sr2 = _call(
        _tc_conv_score,
        (sds((NP, D), f32), sds((NP, 1), f32), sds((NP, 1), f32)),
        rsp2, y2, dinv2, b2.reshape(1, D), pWn2, pWr2, keep1)
    segp2 = _sc_segsum(srcp, dstp, t2.reshape(NP))
    _, _, x2 = _call(
        _tc_pool(K2),
        (sds((NP, D), f32), sds((NP, 1), f32), sds((1, 2 * D), f32)),
        segp2.reshape(NC, NP, 1), sr2, pb2.reshape(1, 1), keep1, h2)

    return _call(_tc_head, sds((1, C), f32), x1, x2, fW1, fb1.reshape(1, D),
                 fW2, fb2.reshape(1, C))
